# baseline (speedup 1.0000x reference)
import functools

import jax
import jax.numpy as jnp
from jax import lax
from jax.experimental import pallas as pl
from jax.experimental.pallas import tpu as pltpu
from jax.experimental.pallas import tpu_sc as plsc

_ROWS = 8192
_DIM = 1024
_NC = 2
_NS = 16
_RPW = _ROWS // (_NC * _NS)
_CH = 64
_NCHUNK = _RPW // _CH
_NBUF = 2


@functools.partial(
    pl.kernel,
    out_type=jax.ShapeDtypeStruct((_ROWS, _DIM), jnp.float32),
    mesh=plsc.VectorSubcoreMesh(core_axis_name="c", subcore_axis_name="s"),
    scratch_types=[
        pltpu.VMEM((_NBUF, _CH, _DIM), jnp.float32),
        pltpu.SemaphoreType.DMA,
        pltpu.SemaphoreType.DMA,
    ],
)
def _sc_copy(w_hbm, out_hbm, buf, in_sem, out_sem):
    wid = lax.axis_index("s") * _NC + lax.axis_index("c")
    base = wid * _RPW

    def in_copy(i):
        return pltpu.make_async_copy(
            w_hbm.at[pl.ds(base + i * _CH, _CH)], buf.at[i % _NBUF], in_sem)

    def out_copy(i):
        return pltpu.make_async_copy(
            buf.at[i % _NBUF], out_hbm.at[pl.ds(base + i * _CH, _CH)], out_sem)

    for i in range(_NBUF - 1):
        in_copy(i).start()
    for i in range(_NCHUNK):
        in_copy(i).wait()
        out_copy(i).start()
        if i + _NBUF - 1 < _NCHUNK:
            if i >= 1:
                out_copy(i - 1).wait()
            in_copy(i + _NBUF - 1).start()
    for i in range(_NCHUNK - _NBUF, _NCHUNK):
        if i >= 0:
            out_copy(i).wait()


def kernel(x, W):
    del x
    return _sc_copy(W)[None]


# speedup vs baseline: 1.0180x; 1.0180x over previous
import functools

import jax
import jax.numpy as jnp
from jax import lax
from jax.experimental import pallas as pl
from jax.experimental.pallas import tpu as pltpu
from jax.experimental.pallas import tpu_sc as plsc

_ROWS = 8192
_DIM = 1024
_NC = 2
_NS = 16
_RPW = _ROWS // (_NC * _NS)
_CH = 16
_NCHUNK = _RPW // _CH
_NBUF = 8


@functools.partial(
    pl.kernel,
    out_type=jax.ShapeDtypeStruct((_ROWS, _DIM), jnp.float32),
    mesh=plsc.VectorSubcoreMesh(core_axis_name="c", subcore_axis_name="s"),
    scratch_types=[
        pltpu.VMEM((_NBUF, _CH, _DIM), jnp.float32),
        pltpu.SemaphoreType.DMA,
        pltpu.SemaphoreType.DMA,
    ],
)
def _sc_copy(w_hbm, out_hbm, buf, in_sem, out_sem):
    wid = lax.axis_index("s") * _NC + lax.axis_index("c")
    base = wid * _RPW

    def in_copy(i):
        return pltpu.make_async_copy(
            w_hbm.at[pl.ds(base + i * _CH, _CH)], buf.at[i % _NBUF], in_sem)

    def out_copy(i):
        return pltpu.make_async_copy(
            buf.at[i % _NBUF], out_hbm.at[pl.ds(base + i * _CH, _CH)], out_sem)

    for i in range(_NBUF - 1):
        in_copy(i).start()
    for i in range(_NCHUNK):
        in_copy(i).wait()
        out_copy(i).start()
        if i + _NBUF - 1 < _NCHUNK:
            if i >= 1:
                out_copy(i - 1).wait()
            in_copy(i + _NBUF - 1).start()
    for i in range(_NCHUNK - _NBUF, _NCHUNK):
        if i >= 0:
            out_copy(i).wait()


def kernel(x, W):
    del x
    return _sc_copy(W)[None]


# speedup vs baseline: 1.0350x; 1.0167x over previous
import functools

import jax
import jax.numpy as jnp
from jax import lax
from jax.experimental import pallas as pl
from jax.experimental.pallas import tpu as pltpu
from jax.experimental.pallas import tpu_sc as plsc

_ROWS = 8192
_DIM = 1024
_NC = 2
_NS = 16
_RPW = _ROWS // (_NC * _NS)
_CH = 32
_NCHUNK = _RPW // _CH
_NBUF = 4


@functools.partial(
    pl.kernel,
    out_type=jax.ShapeDtypeStruct((_ROWS, _DIM), jnp.float32),
    mesh=plsc.VectorSubcoreMesh(core_axis_name="c", subcore_axis_name="s"),
    scratch_types=[
        pltpu.VMEM((_NBUF, _CH, _DIM), jnp.float32),
        pltpu.SemaphoreType.DMA,
        pltpu.SemaphoreType.DMA,
    ],
)
def _sc_copy(w_hbm, out_hbm, buf, in_sem, out_sem):
    wid = lax.axis_index("c") * _NS + lax.axis_index("s")
    base = wid * _RPW

    def in_copy(i):
        return pltpu.make_async_copy(
            w_hbm.at[pl.ds(base + i * _CH, _CH)], buf.at[i % _NBUF], in_sem)

    def out_copy(i):
        return pltpu.make_async_copy(
            buf.at[i % _NBUF], out_hbm.at[pl.ds(base + i * _CH, _CH)], out_sem)

    for i in range(_NBUF - 1):
        in_copy(i).start()
    for i in range(_NCHUNK):
        in_copy(i).wait()
        out_copy(i).start()
        if i + _NBUF - 1 < _NCHUNK:
            if i >= 1:
                out_copy(i - 1).wait()
            in_copy(i + _NBUF - 1).start()
    for i in range(_NCHUNK - _NBUF, _NCHUNK):
        if i >= 0:
            out_copy(i).wait()


def kernel(x, W):
    del x
    return _sc_copy(W)[None]


# speedup vs baseline: 1.0415x; 1.0063x over previous
import functools

import jax
import jax.numpy as jnp
from jax import lax
from jax.experimental import pallas as pl
from jax.experimental.pallas import tpu as pltpu
from jax.experimental.pallas import tpu_sc as plsc

_ROWS = 8192
_DIM = 1024
_NC = 2
_NS = 16
_RPW = _ROWS // (_NC * _NS)
_CH = 32
_NCHUNK = _RPW // _CH
_NBUF = 4


@functools.partial(
    pl.kernel,
    out_type=jax.ShapeDtypeStruct((_ROWS, _DIM), jnp.float32),
    mesh=plsc.VectorSubcoreMesh(core_axis_name="c", subcore_axis_name="s"),
    scratch_types=[pltpu.VMEM((_NBUF, _CH, _DIM), jnp.float32)]
    + [pltpu.SemaphoreType.DMA] * (2 * _NBUF),
)
def _sc_copy(w_hbm, out_hbm, buf, *sems):
    in_sems, out_sems = sems[:_NBUF], sems[_NBUF:]
    wid = lax.axis_index("c") * _NS + lax.axis_index("s")
    base = wid * _RPW

    def in_copy(i):
        return pltpu.make_async_copy(
            w_hbm.at[pl.ds(base + i * _CH, _CH)],
            buf.at[i % _NBUF], in_sems[i % _NBUF])

    def out_copy(i):
        return pltpu.make_async_copy(
            buf.at[i % _NBUF],
            out_hbm.at[pl.ds(base + i * _CH, _CH)], out_sems[i % _NBUF])

    for i in range(min(_NBUF - 1, _NCHUNK)):
        in_copy(i).start()
    for i in range(_NCHUNK):
        in_copy(i).wait()
        out_copy(i).start()
        if i + _NBUF - 1 < _NCHUNK:
            if i >= 1:
                out_copy(i - 1).wait()
            in_copy(i + _NBUF - 1).start()
    for i in range(max(0, _NCHUNK - _NBUF), _NCHUNK):
        out_copy(i).wait()


def kernel(x, W):
    del x
    return _sc_copy(W)[None]
